# baseline (speedup 1.0000x reference)
import functools

import jax
import jax.numpy as jnp
from jax import lax
from jax.experimental import pallas as pl
from jax.experimental.pallas import tpu as pltpu
from jax.experimental.pallas import tpu_sc as plsc

N_NODES = 10000
N_EDGES = 320000
D = 128

NC = 2
NS = 16
NW = NC * NS
CH = 128
KCH = 80
HK = KCH // 2
EPW = KCH * CH
E_PAD = NW * EPW
HP = 10240
ROWS_PER_TILE = HP // NS


def _sc_segment_sum(x, src2d, dst2d):
    mesh = plsc.VectorSubcoreMesh(core_axis_name="c", subcore_axis_name="s")

    @functools.partial(
        pl.kernel,
        mesh=mesh,
        out_type=jax.ShapeDtypeStruct((NC, HP, D), jnp.float32),
        scratch_types=[
            pltpu.VMEM((HK, CH), jnp.int32),
            pltpu.VMEM((HK, CH), jnp.int32),
            pltpu.VMEM((CH, D), jnp.float32),
            pltpu.VMEM((CH, D), jnp.float32),
            pltpu.VMEM_SHARED((HP, D), jnp.float32),
            pltpu.SemaphoreType.DMA,
            pltpu.SemaphoreType.DMA,
        ],
    )
    def k(x_hbm, src_hbm, dst_hbm, out_hbm, src_v, dst_v, rows_v, rows2_v,
          h_sh, sem, sem2):
        c = lax.axis_index("c")
        s = lax.axis_index("s")
        wid = s * NC + c

        def zrow(i, _):
            def zcol(j, _):
                rows_v[i, pl.ds(j * 16, 16)] = jnp.zeros((16,), jnp.float32)
                return 0
            return lax.fori_loop(0, D // 16, zcol, 0)
        lax.fori_loop(0, CH, zrow, 0)

        base = s * ROWS_PER_TILE

        def zblk(t, _):
            pltpu.sync_copy(rows_v, h_sh.at[pl.ds(base + t * CH, CH)])
            return 0
        lax.fori_loop(0, ROWS_PER_TILE // CH, zblk, 0)
        plsc.subcore_barrier()

        for half in range(2):
            pltpu.sync_copy(src_hbm.at[wid, pl.ds(half * HK, HK)], src_v)
            pltpu.sync_copy(dst_hbm.at[wid, pl.ds(half * HK, HK)], dst_v)

            def body(t, _):
                j0 = 2 * t
                g0 = pltpu.async_copy(x_hbm.at[src_v.at[j0]], rows_v, sem)
                g1 = pltpu.async_copy(
                    x_hbm.at[src_v.at[j0 + 1]], rows2_v, sem2)
                g0.wait()
                g1.wait()
                pltpu.sync_copy(rows_v, h_sh.at[dst_v.at[j0]], add=True)
                pltpu.sync_copy(rows2_v, h_sh.at[dst_v.at[j0 + 1]], add=True)
                return 0
            lax.fori_loop(0, HK // 2, body, 0)
        plsc.subcore_barrier()

        pltpu.sync_copy(h_sh.at[pl.ds(base, ROWS_PER_TILE)],
                        out_hbm.at[c, pl.ds(base, ROWS_PER_TILE)])

    return k(x, src2d, dst2d)


def _tc_linear(h2, W, b2):
    BLK = 1024

    def body(h_ref, w_ref, b_ref, o_ref):
        hsum = h_ref[0] + h_ref[1]
        o_ref[...] = lax.dot_general(
            hsum, w_ref[...], (((1,), (1,)), ((), ())),
            preferred_element_type=jnp.float32) + b_ref[...]

    return pl.pallas_call(
        body,
        grid=(HP // BLK,),
        in_specs=[
            pl.BlockSpec((NC, BLK, D), lambda i: (0, i, 0)),
            pl.BlockSpec((D, D), lambda i: (0, 0)),
            pl.BlockSpec((1, D), lambda i: (0, 0)),
        ],
        out_specs=pl.BlockSpec((BLK, D), lambda i: (i, 0)),
        out_shape=jax.ShapeDtypeStruct((HP, D), jnp.float32),
    )(h2, W, b2)


def kernel(x, edge_index, W, b):
    src = edge_index[0].astype(jnp.int32)
    dst = edge_index[1].astype(jnp.int32)
    pad = E_PAD - N_EDGES
    src_p = jnp.concatenate(
        [src, jnp.zeros((pad,), jnp.int32)]).reshape(NW, KCH, CH)
    dst_pad = N_NODES + (jnp.arange(pad, dtype=jnp.int32) % (HP - N_NODES))
    dst_p = jnp.concatenate([dst, dst_pad]).reshape(NW, KCH, CH)
    h2 = _sc_segment_sum(x, src_p, dst_p)
    out = _tc_linear(h2, W, b.reshape(1, D))
    return out[:N_NODES]


# speedup vs baseline: 1.0067x; 1.0067x over previous
import functools

import jax
import jax.numpy as jnp
from jax import lax
from jax.experimental import pallas as pl
from jax.experimental.pallas import tpu as pltpu
from jax.experimental.pallas import tpu_sc as plsc

N_NODES = 10000
N_EDGES = 320000
D = 128

NC = 2
NS = 16
NW = NC * NS
CH = 64
KCH = 160
QK = KCH // 4
EPW = KCH * CH
E_PAD = NW * EPW
HP = 10240
ROWS_PER_TILE = HP // NS


def _sc_segment_sum(x, src2d, dst2d):
    mesh = plsc.VectorSubcoreMesh(core_axis_name="c", subcore_axis_name="s")

    @functools.partial(
        pl.kernel,
        mesh=mesh,
        out_type=jax.ShapeDtypeStruct((NC, HP, D), jnp.float32),
        scratch_types=[
            pltpu.VMEM((QK, CH), jnp.int32),
            pltpu.VMEM((QK, CH), jnp.int32),
            pltpu.VMEM((4, CH, D), jnp.float32),
            pltpu.VMEM_SHARED((HP, D), jnp.float32),
            pltpu.SemaphoreType.DMA,
            pltpu.SemaphoreType.DMA,
            pltpu.SemaphoreType.DMA,
            pltpu.SemaphoreType.DMA,
            pltpu.SemaphoreType.DMA,
            pltpu.SemaphoreType.DMA,
            pltpu.SemaphoreType.DMA,
            pltpu.SemaphoreType.DMA,
        ],
    )
    def k(x_hbm, src_hbm, dst_hbm, out_hbm, src_v, dst_v, rows_v, h_sh,
          gs0, gs1, gs2, gs3, ss0, ss1, ss2, ss3):
        c = lax.axis_index("c")
        s = lax.axis_index("s")
        wid = s * NC + c

        def zrow(i, _):
            def zcol(j, _):
                rows_v[0, i, pl.ds(j * 16, 16)] = jnp.zeros((16,), jnp.float32)
                rows_v[1, i, pl.ds(j * 16, 16)] = jnp.zeros((16,), jnp.float32)
                return 0
            return lax.fori_loop(0, D // 16, zcol, 0)
        lax.fori_loop(0, CH, zrow, 0)

        base = s * ROWS_PER_TILE

        def zblk(t, _):
            pltpu.sync_copy(rows_v.at[0], h_sh.at[pl.ds(base + t * CH, CH)])
            return 0
        lax.fori_loop(0, ROWS_PER_TILE // CH, zblk, 0)
        plsc.subcore_barrier()

        for q in range(4):
            pltpu.sync_copy(src_hbm.at[wid, pl.ds(q * QK, QK)], src_v)
            pltpu.sync_copy(dst_hbm.at[wid, pl.ds(q * QK, QK)], dst_v)

            def it(t, _):
                j0 = 4 * t
                g0 = pltpu.async_copy(
                    x_hbm.at[src_v.at[j0]], rows_v.at[0], gs0)
                g1 = pltpu.async_copy(
                    x_hbm.at[src_v.at[j0 + 1]], rows_v.at[1], gs1)
                g0.wait()
                g1.wait()
                s0 = pltpu.async_copy(
                    rows_v.at[0], h_sh.at[dst_v.at[j0]], ss0, add=True)
                s1 = pltpu.async_copy(
                    rows_v.at[1], h_sh.at[dst_v.at[j0 + 1]], ss1, add=True)
                g2 = pltpu.async_copy(
                    x_hbm.at[src_v.at[j0 + 2]], rows_v.at[2], gs2)
                g3 = pltpu.async_copy(
                    x_hbm.at[src_v.at[j0 + 3]], rows_v.at[3], gs3)
                g2.wait()
                g3.wait()
                s2 = pltpu.async_copy(
                    rows_v.at[2], h_sh.at[dst_v.at[j0 + 2]], ss2, add=True)
                s3 = pltpu.async_copy(
                    rows_v.at[3], h_sh.at[dst_v.at[j0 + 3]], ss3, add=True)
                s0.wait()
                s1.wait()
                s2.wait()
                s3.wait()
                return 0
            lax.fori_loop(0, QK // 4, it, 0)
        plsc.subcore_barrier()

        pltpu.sync_copy(h_sh.at[pl.ds(base, ROWS_PER_TILE)],
                        out_hbm.at[c, pl.ds(base, ROWS_PER_TILE)])

    return k(x, src2d, dst2d)


def _tc_linear(h2, W, b2):
    BLK = 1024

    def body(h_ref, w_ref, b_ref, o_ref):
        hsum = h_ref[0] + h_ref[1]
        o_ref[...] = lax.dot_general(
            hsum, w_ref[...], (((1,), (1,)), ((), ())),
            preferred_element_type=jnp.float32) + b_ref[...]

    return pl.pallas_call(
        body,
        grid=(HP // BLK,),
        in_specs=[
            pl.BlockSpec((NC, BLK, D), lambda i: (0, i, 0)),
            pl.BlockSpec((D, D), lambda i: (0, 0)),
            pl.BlockSpec((1, D), lambda i: (0, 0)),
        ],
        out_specs=pl.BlockSpec((BLK, D), lambda i: (i, 0)),
        out_shape=jax.ShapeDtypeStruct((HP, D), jnp.float32),
    )(h2, W, b2)


def kernel(x, edge_index, W, b):
    src = edge_index[0].astype(jnp.int32)
    dst = edge_index[1].astype(jnp.int32)
    pad = E_PAD - N_EDGES
    src_p = jnp.concatenate(
        [src, jnp.zeros((pad,), jnp.int32)]).reshape(NW, KCH, CH)
    dst_pad = N_NODES + (jnp.arange(pad, dtype=jnp.int32) % (HP - N_NODES))
    dst_p = jnp.concatenate([dst, dst_pad]).reshape(NW, KCH, CH)
    h2 = _sc_segment_sum(x, src_p, dst_p)
    out = _tc_linear(h2, W, b.reshape(1, D))
    return out[:N_NODES]


# speedup vs baseline: 1.5280x; 1.5178x over previous
import functools

import jax
import jax.numpy as jnp
from jax import lax
from jax.experimental import pallas as pl
from jax.experimental.pallas import tpu as pltpu
from jax.experimental.pallas import tpu_sc as plsc

N_NODES = 10000
N_EDGES = 320000
D = 128

NC = 2
NS = 16
NW = NC * NS
CH = 128
KCH = 79
EPW = KCH * CH
E_PAD = NW * EPW
HP = 10240
ROWS_PER_TILE = HP // NS


def _sc_segment_sum(x, src2d, dst2d):
    mesh = plsc.VectorSubcoreMesh(core_axis_name="c", subcore_axis_name="s")

    @functools.partial(
        pl.kernel,
        mesh=mesh,
        out_type=jax.ShapeDtypeStruct((NC, HP, D), jnp.float32),
        scratch_types=[
            pltpu.VMEM((KCH, CH), jnp.int32),
            pltpu.VMEM((KCH, CH), jnp.int32),
            pltpu.VMEM((CH, D), jnp.float32),
            pltpu.VMEM_SHARED((HP, D), jnp.float32),
            pltpu.SemaphoreType.DMA,
        ],
    )
    def k(x_hbm, src_hbm, dst_hbm, out_hbm, src_v, dst_v, rows_v, h_sh, sem):
        c = lax.axis_index("c")
        s = lax.axis_index("s")
        wid = s * NC + c

        def zrow(i, _):
            def zcol(j, _):
                rows_v[i, pl.ds(j * 16, 16)] = jnp.zeros((16,), jnp.float32)
                return 0
            return lax.fori_loop(0, D // 16, zcol, 0)
        lax.fori_loop(0, CH, zrow, 0)

        base = s * ROWS_PER_TILE

        def zblk(t, _):
            pltpu.sync_copy(rows_v, h_sh.at[pl.ds(base + t * CH, CH)])
            return 0
        lax.fori_loop(0, ROWS_PER_TILE // CH, zblk, 0)
        plsc.subcore_barrier()

        pltpu.sync_copy(src_hbm.at[wid], src_v)
        pltpu.sync_copy(dst_hbm.at[wid], dst_v)

        def body(j, _):
            pltpu.async_copy(x_hbm.at[src_v.at[j]], rows_v, sem).wait()
            pltpu.sync_copy(rows_v, h_sh.at[dst_v.at[j]], add=True)
            return 0
        lax.fori_loop(0, KCH, body, 0)
        plsc.subcore_barrier()

        pltpu.sync_copy(h_sh.at[pl.ds(base, ROWS_PER_TILE)],
                        out_hbm.at[c, pl.ds(base, ROWS_PER_TILE)])

    return k(x, src2d, dst2d)


def _tc_linear(h2, W, b2):
    BLK = 1024

    def body(h_ref, w_ref, b_ref, o_ref):
        hsum = h_ref[0] + h_ref[1]
        o_ref[...] = lax.dot_general(
            hsum, w_ref[...], (((1,), (1,)), ((), ())),
            preferred_element_type=jnp.float32) + b_ref[...]

    return pl.pallas_call(
        body,
        grid=(HP // BLK,),
        in_specs=[
            pl.BlockSpec((NC, BLK, D), lambda i: (0, i, 0)),
            pl.BlockSpec((D, D), lambda i: (0, 0)),
            pl.BlockSpec((1, D), lambda i: (0, 0)),
        ],
        out_specs=pl.BlockSpec((BLK, D), lambda i: (i, 0)),
        out_shape=jax.ShapeDtypeStruct((HP, D), jnp.float32),
    )(h2, W, b2)


def kernel(x, edge_index, W, b):
    src = edge_index[0].astype(jnp.int32)
    dst = edge_index[1].astype(jnp.int32)
    pad = E_PAD - N_EDGES
    src_p = jnp.concatenate(
        [src, jnp.zeros((pad,), jnp.int32)]).reshape(NW, KCH, CH)
    dst_pad = N_NODES + (jnp.arange(pad, dtype=jnp.int32) % (HP - N_NODES))
    dst_p = jnp.concatenate([dst, dst_pad]).reshape(NW, KCH, CH)
    h2 = _sc_segment_sum(x, src_p, dst_p)
    out = _tc_linear(h2, W, b.reshape(1, D))
    return out[:N_NODES]
